# baseline (speedup 1.0000x reference)
import functools

import jax
import jax.numpy as jnp
from jax.experimental import pallas as pl
from jax.experimental.pallas import tpu as pltpu
from jax.experimental.pallas import tpu_sc as plsc

_H = 2048
_D = 512
_NH = 16
_TPAD = 2 * _H
_NBANK = 8
_NTILES = 32
_FIRE = 8


def _mlp_banks_kernel(c_ref, w0_ref, b0_ref, w1t_ref, o_ref):
    r = jnp.maximum(w0_ref[...] * c_ref[...] + b0_ref[...], 0.0)
    t = jax.lax.dot_general(
        w1t_ref[...], r, (((1,), (0,)), ((), ())),
        preferred_element_type=jnp.float32,
        precision=jax.lax.Precision.HIGHEST,
    )
    t = 16.0 * jax.nn.sigmoid(t)
    for b in range(_NBANK):
        o_ref[b] = pltpu.roll(t, (_TPAD - b) % _TPAD, axis=1)


def _sc_expand(banks_hbm, out_hbm, banks_sp, sem_row):
    c = jax.lax.axis_index("c")
    s = jax.lax.axis_index("s")
    wid = s * 2 + c

    @pl.when(s == 0)
    def _stage_banks():
        pltpu.sync_copy(banks_hbm, banks_sp)

    plsc.subcore_barrier()

    rows_per_tile = (_NH * _H) // _NTILES

    def body(g, carry):
        descs = []
        for j in range(_FIRE):
            row = wid * rows_per_tile + g * _FIRE + j
            n = row // _H
            i = row % _H
            start = (_H - 1) - i
            b = start % _NBANK
            a = pl.multiple_of(start - b, 8)
            descs.append(pltpu.async_copy(
                banks_sp.at[b, n, pl.ds(a, _H)],
                out_hbm.at[n, i], sem_row))
        for d in descs:
            d.wait()
        return carry

    jax.lax.fori_loop(0, rows_per_tile // _FIRE, body, 0)


def kernel(h, h2, bc, W0, b0, W1):
    f32 = jnp.float32
    coords_open = jnp.arange(-(_H - 1), _H, dtype=f32) / (h - 1)
    periodic_parts = jnp.concatenate([
        jnp.arange(1, _H // 2 + 1, dtype=f32),
        jnp.arange(-(_H // 2 - 1), _H // 2 + 1, dtype=f32),
        jnp.arange(-(_H // 2 - 1), 0, dtype=f32),
    ]) / (h - 1)
    pad_len = 2 * _H - 1 - periodic_parts.shape[0]
    coords_periodic = jnp.concatenate(
        [periodic_parts, jnp.zeros(pad_len, dtype=f32)])
    rel = jnp.where(bc == 1, coords_periodic, coords_open)
    c_pad = jnp.concatenate([rel, jnp.zeros(1, dtype=f32)]).reshape(1, _TPAD)

    w0c = W0.reshape(1, _D).T
    b0c = b0.reshape(_D, 1)
    w1t = W1.reshape(_D, _NH).T

    banks = pl.pallas_call(
        _mlp_banks_kernel,
        in_specs=[
            pl.BlockSpec((1, _TPAD), lambda: (0, 0)),
            pl.BlockSpec((_D, 1), lambda: (0, 0)),
            pl.BlockSpec((_D, 1), lambda: (0, 0)),
            pl.BlockSpec((_NH, _D), lambda: (0, 0)),
        ],
        out_specs=pl.BlockSpec((_NBANK, _NH, _TPAD), lambda: (0, 0, 0)),
        out_shape=jax.ShapeDtypeStruct((_NBANK, _NH, _TPAD), f32),
    )(c_pad, w0c, b0c, w1t)

    expand = functools.partial(
        pl.kernel,
        out_type=jax.ShapeDtypeStruct((_NH, _H, _H), f32),
        mesh=plsc.VectorSubcoreMesh(core_axis_name="c", subcore_axis_name="s"),
        compiler_params=pltpu.CompilerParams(use_tc_tiling_on_sc=False),
        scratch_types=[
            pltpu.VMEM_SHARED((_NBANK, _NH, _TPAD), f32),
            pltpu.SemaphoreType.DMA,
        ],
    )(_sc_expand)
    out = expand(banks)
    return out[None]


# speedup vs baseline: 4.6360x; 4.6360x over previous
import jax
import jax.numpy as jnp
from jax.experimental import pallas as pl
from jax.experimental.pallas import tpu as pltpu

_H = 2048
_D = 512
_NH = 16
_TPAD = 2 * _H
_BLK_ROWS = 512
_SLAB = _H + _BLK_ROWS


def _mlp_kernel(c_ref, w0_ref, b0_ref, w1t_ref, o_ref):
    r = jnp.maximum(w0_ref[...] * c_ref[...] + b0_ref[...], 0.0)
    t = jax.lax.dot_general(
        w1t_ref[...], r, (((1,), (0,)), ((), ())),
        preferred_element_type=jnp.float32,
        precision=jax.lax.Precision.HIGHEST,
    )
    o_ref[...] = 16.0 * jax.nn.sigmoid(t)


def _toeplitz_kernel(t_ref, o_ref):
    i0 = pl.program_id(1) * _BLK_ROWS
    slab = t_ref[0, :, pl.ds(_H - _BLK_ROWS - i0, _SLAB)]
    x = jnp.broadcast_to(slab, (_BLK_ROWS, _SLAB))
    y = pltpu.roll(x, _SLAB - (_BLK_ROWS - 1), axis=1, stride=1,
                   stride_axis=0)
    o_ref[...] = y[None, :, :_H]


def kernel(h, h2, bc, W0, b0, W1):
    f32 = jnp.float32
    coords_open = jnp.arange(-(_H - 1), _H, dtype=f32) / (h - 1)
    periodic_parts = jnp.concatenate([
        jnp.arange(1, _H // 2 + 1, dtype=f32),
        jnp.arange(-(_H // 2 - 1), _H // 2 + 1, dtype=f32),
        jnp.arange(-(_H // 2 - 1), 0, dtype=f32),
    ]) / (h - 1)
    pad_len = 2 * _H - 1 - periodic_parts.shape[0]
    coords_periodic = jnp.concatenate(
        [periodic_parts, jnp.zeros(pad_len, dtype=f32)])
    rel = jnp.where(bc == 1, coords_periodic, coords_open)
    c_pad = jnp.concatenate([rel, jnp.zeros(1, dtype=f32)]).reshape(1, _TPAD)

    w0c = W0.reshape(1, _D).T
    b0c = b0.reshape(_D, 1)
    w1t = W1.reshape(_D, _NH).T

    t_pad = pl.pallas_call(
        _mlp_kernel,
        in_specs=[
            pl.BlockSpec((1, _TPAD), lambda: (0, 0)),
            pl.BlockSpec((_D, 1), lambda: (0, 0)),
            pl.BlockSpec((_D, 1), lambda: (0, 0)),
            pl.BlockSpec((_NH, _D), lambda: (0, 0)),
        ],
        out_specs=pl.BlockSpec((_NH, _TPAD), lambda: (0, 0)),
        out_shape=jax.ShapeDtypeStruct((_NH, _TPAD), f32),
    )(c_pad, w0c, b0c, w1t)
    t_pad = t_pad.reshape(_NH, 1, _TPAD)

    out = pl.pallas_call(
        _toeplitz_kernel,
        grid=(_NH, _H // _BLK_ROWS),
        in_specs=[pl.BlockSpec((1, 1, _TPAD), lambda n, ib: (n, 0, 0))],
        out_specs=pl.BlockSpec((1, _BLK_ROWS, _H), lambda n, ib: (n, ib, 0)),
        out_shape=jax.ShapeDtypeStruct((_NH, _H, _H), f32),
        compiler_params=pltpu.CompilerParams(
            dimension_semantics=("parallel", "parallel")),
    )(t_pad)
    return out[None]


# speedup vs baseline: 4.8027x; 1.0360x over previous
import jax
import jax.numpy as jnp
from jax.experimental import pallas as pl
from jax.experimental.pallas import tpu as pltpu

_H = 2048
_D = 512
_NH = 16
_TPAD = 2 * _H
_BLK_ROWS = 512
_SLAB = _H + _BLK_ROWS


def _fused_kernel(c_ref, w0_ref, b0_ref, w1t_ref, o_ref, t_scr):
    n = pl.program_id(0)
    ib = pl.program_id(1)

    @pl.when((n == 0) & (ib == 0))
    def _build_table():
        r = jnp.maximum(w0_ref[...] * c_ref[...] + b0_ref[...], 0.0)
        t = jax.lax.dot_general(
            w1t_ref[...], r, (((1,), (0,)), ((), ())),
            preferred_element_type=jnp.float32,
            precision=jax.lax.Precision.HIGHEST,
        )
        t_scr[...] = 16.0 * jax.nn.sigmoid(t)

    i0 = ib * _BLK_ROWS
    slab = t_scr[pl.ds(n, 1), pl.ds(_H - _BLK_ROWS - i0, _SLAB)]
    x = jnp.broadcast_to(slab, (_BLK_ROWS, _SLAB))
    y = pltpu.roll(x, _SLAB - (_BLK_ROWS - 1), axis=1, stride=1,
                   stride_axis=0)
    o_ref[...] = y[None, :, :_H]


def kernel(h, h2, bc, W0, b0, W1):
    f32 = jnp.float32
    coords_open = jnp.arange(-(_H - 1), _H, dtype=f32) / (h - 1)
    periodic_parts = jnp.concatenate([
        jnp.arange(1, _H // 2 + 1, dtype=f32),
        jnp.arange(-(_H // 2 - 1), _H // 2 + 1, dtype=f32),
        jnp.arange(-(_H // 2 - 1), 0, dtype=f32),
    ]) / (h - 1)
    pad_len = 2 * _H - 1 - periodic_parts.shape[0]
    coords_periodic = jnp.concatenate(
        [periodic_parts, jnp.zeros(pad_len, dtype=f32)])
    rel = jnp.where(bc == 1, coords_periodic, coords_open)
    c_pad = jnp.concatenate([rel, jnp.zeros(1, dtype=f32)]).reshape(1, _TPAD)

    w0c = W0.reshape(1, _D).T
    b0c = b0.reshape(_D, 1)
    w1t = W1.reshape(_D, _NH).T

    out = pl.pallas_call(
        _fused_kernel,
        grid=(_NH, _H // _BLK_ROWS),
        in_specs=[
            pl.BlockSpec((1, _TPAD), lambda n, ib: (0, 0)),
            pl.BlockSpec((_D, 1), lambda n, ib: (0, 0)),
            pl.BlockSpec((_D, 1), lambda n, ib: (0, 0)),
            pl.BlockSpec((_NH, _D), lambda n, ib: (0, 0)),
        ],
        out_specs=pl.BlockSpec((1, _BLK_ROWS, _H), lambda n, ib: (n, ib, 0)),
        out_shape=jax.ShapeDtypeStruct((_NH, _H, _H), f32),
        scratch_shapes=[pltpu.VMEM((_NH, _TPAD), f32)],
        compiler_params=pltpu.CompilerParams(
            dimension_semantics=("arbitrary", "arbitrary")),
    )(c_pad, w0c, b0c, w1t)
    return out[None]
